# baseline (speedup 1.0000x reference)
import functools
import math

import jax
import jax.numpy as jnp
import numpy as np
from jax import lax
from jax.experimental import pallas as pl
from jax.experimental.pallas import tpu as pltpu
from jax.experimental.pallas import tpu_sc as plsc

VOCAB = 1000000
D = 64
MAXLEN = 2048
B = 4096
S = 200
EPS = 1e-12

NC = 2
NS = 16
NW = NC * NS
N = B * S
PER_W = N // NW
CHUNK = 128
N_CHUNKS = PER_W // CHUNK


def _pe_over_8():
    position = np.arange(0, S, dtype=np.float32)[:, None]
    div_term = np.exp(
        np.arange(0, D, 2, dtype=np.float32) * (-math.log(10000.0) / D))
    pe = np.zeros((S, D), dtype=np.float32)
    pe[:, 0::2] = np.sin(position * div_term)
    pe[:, 1::2] = np.cos(position * div_term)
    return pe / 8.0


_PE8 = _pe_over_8()


def _sc_gather(ids_3d, W):
    mesh = plsc.VectorSubcoreMesh(core_axis_name="c", subcore_axis_name="s")

    @functools.partial(
        pl.kernel,
        mesh=mesh,
        compiler_params=pltpu.CompilerParams(use_tc_tiling_on_sc=False),
        out_type=jax.ShapeDtypeStruct((N, D), jnp.float32),
        scratch_types=[
            pltpu.VMEM((N_CHUNKS, CHUNK), jnp.int32),
            pltpu.VMEM((CHUNK, D), jnp.float32),
            pltpu.VMEM((CHUNK, D), jnp.float32),
            pltpu.SemaphoreType.DMA,
            pltpu.SemaphoreType.DMA,
        ],
    )
    def k(ids_hbm, w_hbm, out_hbm, idx_v, rows0, rows1, sem0, sem1):
        wid = lax.axis_index("s") * NC + lax.axis_index("c")
        base = wid * PER_W
        pltpu.sync_copy(ids_hbm.at[wid], idx_v)

        @pl.loop(0, N_CHUNKS, step=2)
        def _(c):
            cpa = pltpu.async_copy(w_hbm.at[idx_v.at[c]], rows0, sem0)
            cpb = pltpu.async_copy(w_hbm.at[idx_v.at[c + 1]], rows1, sem1)
            cpa.wait()
            pltpu.sync_copy(rows0, out_hbm.at[pl.ds(base + c * CHUNK, CHUNK)])
            cpb.wait()
            pltpu.sync_copy(
                rows1, out_hbm.at[pl.ds(base + (c + 1) * CHUNK, CHUNK)])

    return k(ids_3d, W)


BB = 64


def _ln_body(g_ref, pe_ref, gam_ref, bet_ref, o_ref):
    x = g_ref[...] + pe_ref[...][None, :, :]
    mu = jnp.mean(x, axis=-1, keepdims=True)
    xc = x - mu
    var = jnp.mean(xc * xc, axis=-1, keepdims=True)
    y = xc * lax.rsqrt(var + EPS / 64.0)
    o_ref[...] = y * gam_ref[...][None, :, :] + bet_ref[...][None, :, :]


def _tc_layernorm(g, pe8, gamma, beta):
    return pl.pallas_call(
        _ln_body,
        grid=(B // BB,),
        in_specs=[
            pl.BlockSpec((BB, S, D), lambda i: (i, 0, 0)),
            pl.BlockSpec((S, D), lambda i: (0, 0)),
            pl.BlockSpec((1, D), lambda i: (0, 0)),
            pl.BlockSpec((1, D), lambda i: (0, 0)),
        ],
        out_specs=pl.BlockSpec((BB, S, D), lambda i: (i, 0, 0)),
        out_shape=jax.ShapeDtypeStruct((B, S, D), jnp.float32),
    )(g, pe8, gamma, beta)


def kernel(input_ids, W, gamma, beta):
    ids_3d = input_ids.reshape(NW, N_CHUNKS, CHUNK).astype(jnp.int32)
    g = _sc_gather(ids_3d, W)
    g = g.reshape(B, S, D)
    pe8 = jnp.asarray(_PE8)
    return _tc_layernorm(g, pe8, gamma.reshape(1, D), beta.reshape(1, D))
